# Initial kernel scaffold; baseline (speedup 1.0000x reference)
import jax
import jax.numpy as jnp
from jax.experimental import pallas as pl


def kernel(x, W1g, b1g, W2g, b2g, W1n, b1n, W2n, b2n):
    raise NotImplementedError("write your pallas kernel here")


# speedup vs baseline: 22.8481x; 22.8481x over previous
import functools

import jax
import jax.numpy as jnp
from jax.experimental import pallas as pl
from jax.experimental.pallas import tpu as pltpu

B, N, D, H = 256, 512, 256, 256
BB = 8


def _graph_summary_kernel(x_ref, w1g_ref, b1g_ref, w2gt_ref, b2g_ref,
                          w1n_ref, b1n_ref, w2n_ref, b2n_ref, out_ref):
    xb = x_ref[...]

    a1 = jnp.dot(xb, w1g_ref[...], preferred_element_type=jnp.float32)
    a1 = a1 + b1g_ref[...]
    a1 = a1 * jax.nn.sigmoid(a1)
    gate = jnp.sum(a1 * w2gt_ref[...], axis=1, keepdims=True) + b2g_ref[...]

    a2 = jnp.dot(xb, w1n_ref[...], preferred_element_type=jnp.float32)
    a2 = a2 + b1n_ref[...]
    a2 = a2 * jax.nn.sigmoid(a2)
    h = jnp.dot(a2, w2n_ref[...], preferred_element_type=jnp.float32)
    h = h + b2n_ref[...]

    for b in range(BB):
        g = gate[b * N:(b + 1) * N, :]
        m = jnp.max(g)
        e = jnp.exp(g - m)
        denom = jnp.sum(e)
        pooled = jnp.sum(e * h[b * N:(b + 1) * N, :], axis=0, keepdims=True)
        out_ref[b:b + 1, :] = pooled / (denom + 1e-16)


@jax.jit
def kernel(x, W1g, b1g, W2g, b2g, W1n, b1n, W2n, b2n):
    flat = x.reshape(B * N, D)
    w2gt = W2g.reshape(1, H)
    grid = (B // BB,)
    full = lambda i: (0, 0)
    out = pl.pallas_call(
        _graph_summary_kernel,
        grid=grid,
        in_specs=[
            pl.BlockSpec((BB * N, D), lambda i: (i, 0)),
            pl.BlockSpec((D, H), full),
            pl.BlockSpec((1, H), full),
            pl.BlockSpec((1, D), full),
            pl.BlockSpec((1, 1), full),
            pl.BlockSpec((D, H), full),
            pl.BlockSpec((1, H), full),
            pl.BlockSpec((H, H), full),
            pl.BlockSpec((1, H), full),
        ],
        out_specs=pl.BlockSpec((BB, H), lambda i: (i, 0)),
        out_shape=jax.ShapeDtypeStruct((B, H), jnp.float32),
        compiler_params=pltpu.CompilerParams(
            dimension_semantics=("arbitrary",),
        ),
    )(flat, W1g, b1g.reshape(1, H), w2gt, b2g.reshape(1, 1),
      W1n, b1n.reshape(1, H), W2n, b2n.reshape(1, H))
    return out


# speedup vs baseline: 25.5904x; 1.1200x over previous
import functools

import jax
import jax.numpy as jnp
from jax.experimental import pallas as pl
from jax.experimental.pallas import tpu as pltpu

B, N, D, H = 256, 512, 256, 256
BB = 8


def _graph_summary_kernel(x_ref, w1g_ref, b1g_ref, w2gt_ref, b2g_ref,
                          w1n_ref, b1n_ref, w2n_ref, b2n_ref, out_ref):
    xb = x_ref[...]

    a1 = jnp.dot(xb, w1g_ref[...], preferred_element_type=jnp.float32)
    a1 = a1 + b1g_ref[...]
    a1 = (0.5 * a1) * (1.0 + jnp.tanh(0.5 * a1))
    gate = jnp.sum(a1 * w2gt_ref[...], axis=1, keepdims=True) + b2g_ref[...]

    a2 = jnp.dot(xb, w1n_ref[...], preferred_element_type=jnp.float32)
    a2 = a2 + b1n_ref[...]
    a2 = (0.5 * a2) * (1.0 + jnp.tanh(0.5 * a2))
    h = jnp.dot(a2, w2n_ref[...], preferred_element_type=jnp.float32)
    h = h + b2n_ref[...]

    for b in range(BB):
        g = gate[b * N:(b + 1) * N, :]
        m = jnp.max(g)
        e = jnp.exp(g - m)
        denom = jnp.sum(e)
        pooled = jax.lax.dot_general(
            e, h[b * N:(b + 1) * N, :],
            (((0,), (0,)), ((), ())),
            preferred_element_type=jnp.float32)
        out_ref[b:b + 1, :] = pooled / (denom + 1e-16)


@jax.jit
def kernel(x, W1g, b1g, W2g, b2g, W1n, b1n, W2n, b2n):
    flat = x.reshape(B * N, D)
    w2gt = W2g.reshape(1, H)
    grid = (B // BB,)
    full = lambda i: (0, 0)
    out = pl.pallas_call(
        _graph_summary_kernel,
        grid=grid,
        in_specs=[
            pl.BlockSpec((BB * N, D), lambda i: (i, 0)),
            pl.BlockSpec((D, H), full),
            pl.BlockSpec((1, H), full),
            pl.BlockSpec((1, D), full),
            pl.BlockSpec((1, 1), full),
            pl.BlockSpec((D, H), full),
            pl.BlockSpec((1, H), full),
            pl.BlockSpec((H, H), full),
            pl.BlockSpec((1, H), full),
        ],
        out_specs=pl.BlockSpec((BB, H), lambda i: (i, 0)),
        out_shape=jax.ShapeDtypeStruct((B, H), jnp.float32),
        compiler_params=pltpu.CompilerParams(
            dimension_semantics=("arbitrary",),
        ),
    )(flat, W1g, b1g.reshape(1, H), w2gt, b2g.reshape(1, 1),
      W1n, b1n.reshape(1, H), W2n, b2n.reshape(1, H))
    return out


# speedup vs baseline: 27.1979x; 1.0628x over previous
import jax
import jax.numpy as jnp
from jax.experimental import pallas as pl
from jax.experimental.pallas import tpu as pltpu

B, N, D, H = 256, 512, 256, 256
BB = 8


def _graph_summary_kernel(x_ref, w1g_ref, b1g_ref, w2gt_ref,
                          w1n_ref, b1n_ref, w2n_ref, b2n_ref, out_ref):
    xb = x_ref[...]

    z1 = jnp.dot(xb, w1g_ref[...], preferred_element_type=jnp.float32)
    z1 = z1 + b1g_ref[...]
    a1 = z1 * (1.0 + jnp.tanh(z1))
    gate = jnp.sum(a1 * w2gt_ref[...], axis=1, keepdims=True)

    z2 = jnp.dot(xb, w1n_ref[...], preferred_element_type=jnp.float32)
    z2 = z2 + b1n_ref[...]
    a2 = z2 * (1.0 + jnp.tanh(z2))
    h = jnp.dot(a2, w2n_ref[...], preferred_element_type=jnp.float32)

    for b in range(BB):
        g = gate[b * N:(b + 1) * N, :]
        m = jnp.max(g)
        e = jnp.exp(g - m)
        denom = jnp.sum(e)
        pooled = jax.lax.dot_general(
            e, h[b * N:(b + 1) * N, :],
            (((0,), (0,)), ((), ())),
            preferred_element_type=jnp.float32)
        out_ref[b:b + 1, :] = pooled / (denom + 1e-16) + b2n_ref[...]


@jax.jit
def kernel(x, W1g, b1g, W2g, b2g, W1n, b1n, W2n, b2n):
    flat = x.reshape(B * N, D)
    grid = (B // BB,)
    full = lambda i: (0, 0)
    out = pl.pallas_call(
        _graph_summary_kernel,
        grid=grid,
        in_specs=[
            pl.BlockSpec((BB * N, D), lambda i: (i, 0)),
            pl.BlockSpec((D, H), full),
            pl.BlockSpec((1, H), full),
            pl.BlockSpec((1, D), full),
            pl.BlockSpec((D, H), full),
            pl.BlockSpec((1, H), full),
            pl.BlockSpec((H, H), full),
            pl.BlockSpec((1, H), full),
        ],
        out_specs=pl.BlockSpec((BB, H), lambda i: (i, 0)),
        out_shape=jax.ShapeDtypeStruct((B, H), jnp.float32),
        compiler_params=pltpu.CompilerParams(
            dimension_semantics=("arbitrary",),
        ),
    )(flat, 0.5 * W1g, (0.5 * b1g).reshape(1, H), W2g.reshape(1, H),
      0.5 * W1n, (0.5 * b1n).reshape(1, H), W2n, b2n.reshape(1, H))
    return out


# speedup vs baseline: 29.1754x; 1.0727x over previous
import jax
import jax.numpy as jnp
from jax.experimental import pallas as pl
from jax.experimental.pallas import tpu as pltpu

B, N, D, H = 256, 512, 256, 256
BB = 8


def _graph_summary_kernel(x_ref, w1_ref, b1_ref, w2gt_ref,
                          w2n_ref, b2n_ref, out_ref):
    xb = x_ref[...]

    z = jnp.dot(xb, w1_ref[...], preferred_element_type=jnp.float32)
    z = z + b1_ref[...]
    a = z * (1.0 + jnp.tanh(z))
    a1 = a[:, :H]
    a2 = a[:, H:]
    gate = jnp.sum(a1 * w2gt_ref[...], axis=1, keepdims=True)

    rows = []
    for b in range(BB):
        g = gate[b * N:(b + 1) * N, :]
        m = jnp.max(g)
        e = jnp.exp(g - m)
        denom = jnp.sum(e)
        pooled = jax.lax.dot_general(
            e, a2[b * N:(b + 1) * N, :],
            (((0,), (0,)), ((), ())),
            preferred_element_type=jnp.float32)
        rows.append(pooled / (denom + 1e-16))
    pooled_all = jnp.concatenate(rows, axis=0)
    out_ref[...] = jnp.dot(pooled_all, w2n_ref[...],
                           preferred_element_type=jnp.float32) + b2n_ref[...]


@jax.jit
def kernel(x, W1g, b1g, W2g, b2g, W1n, b1n, W2n, b2n):
    flat = x.reshape(B * N, D)
    W1 = jnp.concatenate([0.5 * W1g, 0.5 * W1n], axis=1)
    b1 = jnp.concatenate([0.5 * b1g, 0.5 * b1n]).reshape(1, 2 * H)
    grid = (B // BB,)
    full = lambda i: (0, 0)
    out = pl.pallas_call(
        _graph_summary_kernel,
        grid=grid,
        in_specs=[
            pl.BlockSpec((BB * N, D), lambda i: (i, 0)),
            pl.BlockSpec((D, 2 * H), full),
            pl.BlockSpec((1, 2 * H), full),
            pl.BlockSpec((1, H), full),
            pl.BlockSpec((H, H), full),
            pl.BlockSpec((1, H), full),
        ],
        out_specs=pl.BlockSpec((BB, H), lambda i: (i, 0)),
        out_shape=jax.ShapeDtypeStruct((B, H), jnp.float32),
        compiler_params=pltpu.CompilerParams(
            dimension_semantics=("arbitrary",),
        ),
    )(flat, W1, b1, W2g.reshape(1, H),
      W2n, b2n.reshape(1, H))
    return out


# speedup vs baseline: 29.8858x; 1.0243x over previous
import jax
import jax.numpy as jnp
from jax.experimental import pallas as pl
from jax.experimental.pallas import tpu as pltpu

B, N, D, H = 256, 512, 256, 256
BB = 16


def _graph_summary_kernel(x_ref, w1_ref, b1_ref, w2gt_ref,
                          w2n_ref, b2n_ref, out_ref):
    xb = x_ref[...]

    z = jnp.dot(xb, w1_ref[...], preferred_element_type=jnp.float32)
    z = z + b1_ref[...]
    a = z * (1.0 + jnp.tanh(z))
    a1 = a[:, :H]
    a2 = a[:, H:]
    gate = jnp.sum(a1 * w2gt_ref[...], axis=1, keepdims=True)

    rows = []
    for b in range(BB):
        g = gate[b * N:(b + 1) * N, :]
        m = jnp.max(g)
        e = jnp.exp(g - m)
        denom = jnp.sum(e)
        pooled = jax.lax.dot_general(
            e, a2[b * N:(b + 1) * N, :],
            (((0,), (0,)), ((), ())),
            preferred_element_type=jnp.float32)
        rows.append(pooled / (denom + 1e-16))
    pooled_all = jnp.concatenate(rows, axis=0)
    out_ref[...] = jnp.dot(pooled_all, w2n_ref[...],
                           preferred_element_type=jnp.float32) + b2n_ref[...]


@jax.jit
def kernel(x, W1g, b1g, W2g, b2g, W1n, b1n, W2n, b2n):
    flat = x.reshape(B * N, D)
    W1 = jnp.concatenate([0.5 * W1g, 0.5 * W1n], axis=1)
    b1 = jnp.concatenate([0.5 * b1g, 0.5 * b1n]).reshape(1, 2 * H)
    grid = (B // BB,)
    full = lambda i: (0, 0)
    out = pl.pallas_call(
        _graph_summary_kernel,
        grid=grid,
        in_specs=[
            pl.BlockSpec((BB * N, D), lambda i: (i, 0)),
            pl.BlockSpec((D, 2 * H), full),
            pl.BlockSpec((1, 2 * H), full),
            pl.BlockSpec((1, H), full),
            pl.BlockSpec((H, H), full),
            pl.BlockSpec((1, H), full),
        ],
        out_specs=pl.BlockSpec((BB, H), lambda i: (i, 0)),
        out_shape=jax.ShapeDtypeStruct((B, H), jnp.float32),
        compiler_params=pltpu.CompilerParams(
            dimension_semantics=("arbitrary",),
        ),
    )(flat, W1, b1, W2g.reshape(1, H),
      W2n, b2n.reshape(1, H))
    return out


# speedup vs baseline: 29.8908x; 1.0002x over previous
import jax
import jax.numpy as jnp
from jax.experimental import pallas as pl
from jax.experimental.pallas import tpu as pltpu

B, N, D, H = 256, 512, 256, 256
BB = 16


def _graph_summary_kernel(x_ref, w1_ref, b1_ref, w2gt_ref,
                          w2n_ref, b2n_ref, out_ref):
    xb = x_ref[...]

    z = jnp.dot(xb, w1_ref[...], preferred_element_type=jnp.float32)
    z = z + b1_ref[...]
    a = z * (1.0 + jnp.tanh(z))
    a1 = a[:, :H]
    a2 = a[:, H:]
    gate = jnp.sum(a1 * w2gt_ref[...], axis=1, keepdims=True)

    rows = []
    for b in range(BB):
        g = gate[b * N:(b + 1) * N, :]
        m = jnp.max(g)
        e = jnp.exp(g - m)
        denom = jnp.sum(e)
        pooled = jax.lax.dot_general(
            e, a2[b * N:(b + 1) * N, :],
            (((0,), (0,)), ((), ())),
            preferred_element_type=jnp.float32)
        rows.append(pooled / (denom + 1e-16))
    pooled_all = jnp.concatenate(rows, axis=0)
    out_ref[...] = jnp.dot(pooled_all, w2n_ref[...],
                           preferred_element_type=jnp.float32) + b2n_ref[...]


@jax.jit
def kernel(x, W1g, b1g, W2g, b2g, W1n, b1n, W2n, b2n):
    flat = x.reshape(B * N, D)
    W1 = jnp.concatenate([0.5 * W1g, 0.5 * W1n], axis=1)
    b1 = jnp.concatenate([0.5 * b1g, 0.5 * b1n]).reshape(1, 2 * H)
    grid = (B // BB,)
    full = lambda i: (0, 0)
    out = pl.pallas_call(
        _graph_summary_kernel,
        grid=grid,
        in_specs=[
            pl.BlockSpec((BB * N, D), lambda i: (i, 0)),
            pl.BlockSpec((D, 2 * H), full),
            pl.BlockSpec((1, 2 * H), full),
            pl.BlockSpec((1, H), full),
            pl.BlockSpec((H, H), full),
            pl.BlockSpec((1, H), full),
        ],
        out_specs=pl.BlockSpec((BB, H), lambda i: (i, 0)),
        out_shape=jax.ShapeDtypeStruct((B, H), jnp.float32),
        compiler_params=pltpu.CompilerParams(
            dimension_semantics=("parallel",),
        ),
    )(flat, W1, b1, W2g.reshape(1, H),
      W2n, b2n.reshape(1, H))
    return out
